# baseline (speedup 1.0000x reference)
import functools

import jax
import jax.numpy as jnp
from jax import lax
from jax.experimental import pallas as pl
from jax.experimental.pallas import tpu as pltpu
from jax.experimental.pallas import tpu_sc as plsc

_CH = 128


@functools.lru_cache(maxsize=None)
def _make_sc_gather(B, DU, DV):
    info = plsc.get_sparse_core_info()
    NC, NS = info.num_cores, info.num_subcores
    NW = NC * NS
    bw = B // NW
    nch = bw // _CH
    mesh = plsc.VectorSubcoreMesh(core_axis_name="c", subcore_axis_name="s")

    @functools.partial(
        pl.kernel,
        mesh=mesh,
        compiler_params=pltpu.CompilerParams(use_tc_tiling_on_sc=False),
        out_type=(
            jax.ShapeDtypeStruct((B, DU), jnp.float32),
            jax.ShapeDtypeStruct((B, DV), jnp.float32),
        ),
        scratch_types=[
            pltpu.VMEM((nch, _CH), jnp.int32),
            pltpu.VMEM((nch, _CH), jnp.int32),
            pltpu.VMEM((bw, DU), jnp.float32),
            pltpu.VMEM((bw, DV), jnp.float32),
            pltpu.SemaphoreType.DMA,
            pltpu.SemaphoreType.DMA,
        ],
    )
    def gather_kernel(ut, uid, vt, vid, u_out, v_out,
                      uidx, vidx, urows, vrows, su, sv):
        wid = lax.axis_index("s") * NC + lax.axis_index("c")
        base = wid * bw
        for c in range(nch):
            pltpu.sync_copy(uid.at[pl.ds(base + c * _CH, _CH)], uidx.at[c])
            pltpu.sync_copy(vid.at[pl.ds(base + c * _CH, _CH)], vidx.at[c])
        ucopies = [
            pltpu.async_copy(ut.at[uidx.at[c]], urows.at[pl.ds(c * _CH, _CH)], su)
            for c in range(nch)
        ]
        vcopies = [
            pltpu.async_copy(vt.at[vidx.at[c]], vrows.at[pl.ds(c * _CH, _CH)], sv)
            for c in range(nch)
        ]
        for c in ucopies:
            c.wait()
        pltpu.sync_copy(urows, u_out.at[pl.ds(base, bw)])
        for c in vcopies:
            c.wait()
        pltpu.sync_copy(vrows, v_out.at[pl.ds(base, bw)])

    return gather_kernel


def _mlp_body(u_ref, v_ref, w1u_ref, w1v_ref, b1_ref, wo_ref, bo_ref, o_ref):
    h = jnp.dot(u_ref[...], w1u_ref[...], preferred_element_type=jnp.float32)
    h = h + jnp.dot(v_ref[...], w1v_ref[...], preferred_element_type=jnp.float32)
    h = jnp.maximum(h + b1_ref[...], 0.0)
    o_ref[...] = jnp.dot(h, wo_ref[...], preferred_element_type=jnp.float32) + bo_ref[...]


def _mlp(u_emb, v_emb, w1u_t, w1v_t, b1, wout_t, bout):
    B, D = u_emb.shape
    H = w1u_t.shape[1]
    O = wout_t.shape[1]
    blk = 2048
    return pl.pallas_call(
        _mlp_body,
        grid=(B // blk,),
        in_specs=[
            pl.BlockSpec((blk, D), lambda i: (i, 0)),
            pl.BlockSpec((blk, D), lambda i: (i, 0)),
            pl.BlockSpec((D, H), lambda i: (0, 0)),
            pl.BlockSpec((D, H), lambda i: (0, 0)),
            pl.BlockSpec((1, H), lambda i: (0, 0)),
            pl.BlockSpec((H, O), lambda i: (0, 0)),
            pl.BlockSpec((1, O), lambda i: (0, 0)),
        ],
        out_specs=pl.BlockSpec((blk, O), lambda i: (i, 0)),
        out_shape=jax.ShapeDtypeStruct((B, O), jnp.float32),
    )(u_emb, v_emb, w1u_t, w1v_t, b1, wout_t, bout)


def kernel(user_id, video_id, user_table, video_table, W1, b1, Wout, bout):
    B = user_id.shape[0]
    DU = user_table.shape[1]
    DV = video_table.shape[1]
    gather = _make_sc_gather(B, DU, DV)
    u_emb, v_emb = gather(
        user_table, user_id.astype(jnp.int32),
        video_table, video_id.astype(jnp.int32),
    )
    w1u_t = W1[:, :DU].T
    w1v_t = W1[:, DU:].T
    return _mlp(u_emb, v_emb, w1u_t, w1v_t, b1[None, :], Wout.T, bout[None, :])
